# Initial kernel scaffold; baseline (speedup 1.0000x reference)
import jax
import jax.numpy as jnp
from jax.experimental import pallas as pl


def kernel(x, batch_ids, y, W, b):
    raise NotImplementedError("write your pallas kernel here")


# speedup vs baseline: 11.7728x; 11.7728x over previous
import functools

import jax
import jax.numpy as jnp
from jax.experimental import pallas as pl
from jax.experimental.pallas import tpu as pltpu

N_NODES = 50000
D_IN = 512
D_OUT = 128
NUM_GRAPHS = 512

CHUNK = 1000
K = N_NODES // CHUNK


def _body(ids_ref, x_ref, w_ref, b_ref, o_ref, acc_ref, cnt_ref):
    k = pl.program_id(0)

    @pl.when(k == 0)
    def _():
        acc_ref[...] = jnp.zeros_like(acc_ref)
        cnt_ref[...] = jnp.zeros_like(cnt_ref)

    ids = ids_ref[0, 0, :].reshape(CHUNK, 1)
    gids = jax.lax.broadcasted_iota(jnp.int32, (1, NUM_GRAPHS), 1)
    one_hot = (ids == gids).astype(jnp.float32)
    acc_ref[...] += jax.lax.dot_general(
        one_hot, x_ref[...], (((0,), (0,)), ((), ())),
        preferred_element_type=jnp.float32)
    cnt_ref[0:1, :] += jnp.sum(one_hot, axis=0, keepdims=True)

    @pl.when(k == K - 1)
    def _():
        counts = jnp.maximum(cnt_ref[0:1, :], 1.0).reshape(NUM_GRAPHS, 1)
        emb = acc_ref[...] / counts
        o_ref[...] = (
            jnp.dot(emb, w_ref[...], preferred_element_type=jnp.float32)
            + b_ref[...]
        )


def kernel(x, batch_ids, y, W, b):
    ids3d = batch_ids.astype(jnp.int32).reshape(K, 1, CHUNK)
    pred = pl.pallas_call(
        _body,
        grid=(K,),
        in_specs=[
            pl.BlockSpec((1, 1, CHUNK), lambda k: (k, 0, 0)),
            pl.BlockSpec((CHUNK, D_IN), lambda k: (k, 0)),
            pl.BlockSpec((D_IN, D_OUT), lambda k: (0, 0)),
            pl.BlockSpec((1, D_OUT), lambda k: (0, 0)),
        ],
        out_specs=pl.BlockSpec((NUM_GRAPHS, D_OUT), lambda k: (0, 0)),
        out_shape=jax.ShapeDtypeStruct((NUM_GRAPHS, D_OUT), jnp.float32),
        scratch_shapes=[
            pltpu.VMEM((NUM_GRAPHS, D_IN), jnp.float32),
            pltpu.VMEM((8, NUM_GRAPHS), jnp.float32),
        ],
    )(ids3d, x, W, b.reshape(1, D_OUT))
    return (pred, y)
